# baseline (speedup 1.0000x reference)
import functools

import jax
import jax.numpy as jnp
from jax import lax
from jax.experimental import pallas as pl
from jax.experimental.pallas import tpu as pltpu
from jax.experimental.pallas import tpu_sc as plsc

N = 10000
E = 320000
D = 128
DE = 4
H = 128

NC, NS, L = 2, 16, 16
NW = NC * NS
EPW = E // NW
CH = 80
NCHUNK = EPW // CH
NPAD = 10240
RPT = NPAD // NS

_mesh = functools.partial(
    plsc.VectorSubcoreMesh, core_axis_name="c", subcore_axis_name="s")


@functools.partial(
    pl.kernel,
    out_type=jax.ShapeDtypeStruct((NC, NPAD), jnp.float32),
    mesh=_mesh(),
    scratch_types=[
        pltpu.VMEM((NCHUNK, CH), jnp.int32),
        pltpu.VMEM((CH,), jnp.float32),
        pltpu.VMEM_SHARED((NPAD,), jnp.float32),
    ],
)
def _sc_degree(dst_hbm, zeros_hbm, out_hbm, didx_all, ones_v, acc_sh):
  c = lax.axis_index("c")
  s = lax.axis_index("s")
  wid = c * NS + s
  for j in range(CH // L):
    ones_v[pl.ds(j * L, L)] = jnp.ones((L,), jnp.float32)
  pltpu.sync_copy(dst_hbm.at[wid], didx_all)
  pltpu.sync_copy(zeros_hbm.at[pl.ds(s * RPT, RPT)],
                  acc_sh.at[pl.ds(s * RPT, RPT)])
  plsc.subcore_barrier()

  def body(i, carry):
    pltpu.sync_copy(ones_v, acc_sh.at[didx_all.at[i]], add=True)
    return carry

  lax.fori_loop(0, NCHUNK, body, 0)
  plsc.subcore_barrier()
  pltpu.sync_copy(acc_sh.at[pl.ds(s * RPT, RPT)],
                  out_hbm.at[c, pl.ds(s * RPT, RPT)])


@functools.partial(
    pl.kernel,
    out_type=jax.ShapeDtypeStruct((NC, NPAD, H), jnp.float32),
    mesh=_mesh(),
    scratch_types=[
        pltpu.VMEM((EPW,), jnp.int32),
        pltpu.VMEM((NCHUNK, CH), jnp.int32),
        pltpu.VMEM((CH, H), jnp.float32),
        pltpu.VMEM((CH, H), jnp.float32),
        pltpu.VMEM_SHARED((NPAD, H), jnp.float32),
        pltpu.SemaphoreType.DMA,
        pltpu.SemaphoreType.DMA,
        pltpu.SemaphoreType.DMA,
        pltpu.SemaphoreType.DMA,
    ],
)
def _sc_segsum(src_hbm, dst_hbm, g_hbm, zeros_hbm, out_hbm,
               sidx_all, didx_all, rows_a, rows_b, acc_sh,
               sga, sgb, ssa, ssb):
  c = lax.axis_index("c")
  s = lax.axis_index("s")
  wid = c * NS + s
  pltpu.sync_copy(src_hbm.at[wid], sidx_all)
  pltpu.sync_copy(dst_hbm.at[wid], didx_all)

  def gat(i, rows, sem):
    return pltpu.make_async_copy(
        g_hbm.at[sidx_all.at[pl.ds(i * CH, CH)]], rows, sem)

  def scat(i, rows, sem):
    pltpu.async_copy(rows, acc_sh.at[didx_all.at[i]], sem, add=True)

  def scatwait(rows, sem):
    pltpu.make_async_copy(rows, acc_sh.at[didx_all.at[0]], sem).wait()

  gat(0, rows_a, sga).start()
  gat(1, rows_b, sgb).start()
  pltpu.sync_copy(zeros_hbm.at[pl.ds(s * RPT, RPT), :],
                  acc_sh.at[pl.ds(s * RPT, RPT), :])
  plsc.subcore_barrier()

  def body(k, carry):
    gat(0, rows_a, sga).wait()
    scat(2 * k, rows_a, ssa)
    gat(0, rows_b, sgb).wait()
    scat(2 * k + 1, rows_b, ssb)
    scatwait(rows_a, ssa)
    gat(2 * k + 2, rows_a, sga).start()
    scatwait(rows_b, ssb)
    gat(2 * k + 3, rows_b, sgb).start()
    return carry

  lax.fori_loop(0, (NCHUNK - 3) // 2, body, 0)
  k0 = NCHUNK - 3
  gat(0, rows_a, sga).wait()
  scat(k0, rows_a, ssa)
  scatwait(rows_a, ssa)
  gat(k0 + 2, rows_a, sga).start()
  gat(0, rows_b, sgb).wait()
  scat(k0 + 1, rows_b, ssb)
  scatwait(rows_b, ssb)
  gat(0, rows_a, sga).wait()
  scat(k0 + 2, rows_a, ssa)
  scatwait(rows_a, ssa)
  plsc.subcore_barrier()
  pltpu.sync_copy(acc_sh.at[pl.ds(s * RPT, RPT), :],
                  out_hbm.at[c, pl.ds(s * RPT, RPT), :])


def _make_edge_combine(ne, ch):
  epw = ne // NW
  nchunk = epw // ch
  assert nchunk % 2 == 1 and nchunk >= 5 and ch % 8 == 0

  @functools.partial(
      pl.kernel,
      out_type=jax.ShapeDtypeStruct((ne, H), jnp.float32),
      mesh=_mesh(),
      scratch_types=[
          pltpu.VMEM((epw,), jnp.int32),
          pltpu.VMEM((epw,), jnp.int32),
          pltpu.VMEM((2, ch, H), jnp.float32),
          pltpu.VMEM((2, ch, H), jnp.float32),
          pltpu.VMEM((ch,), jnp.int32),
          pltpu.VMEM((ch,), jnp.int32),
          pltpu.VMEM_SHARED((NS * 2 * ch, H), jnp.float32),
          pltpu.SemaphoreType.DMA,
          pltpu.SemaphoreType.DMA,
          pltpu.SemaphoreType.DMA,
          pltpu.SemaphoreType.DMA,
          pltpu.SemaphoreType.DMA,
          pltpu.SemaphoreType.DMA,
      ],
  )
  def _combine(src_hbm, dst_hbm, a_hbm, b_hbm, out_hbm,
               sidx_all, didx_all, bufa, bufb, zi0, zi1, zsh,
               sga0, sga1, sgb0, sgb1, sw0, sw1):
    c = lax.axis_index("c")
    s = lax.axis_index("s")
    wid = c * NS + s
    base0 = wid * epw
    pltpu.sync_copy(src_hbm.at[wid], sidx_all)
    pltpu.sync_copy(dst_hbm.at[wid], didx_all)
    offs = list(range(0, ch - L + 1, L))
    if ch % L:
      offs.append(ch - L)
    for off in offs:
      lane = lax.iota(jnp.int32, L) + (s * 2 * ch + off)
      zi0[pl.ds(off, L)] = lane
      zi1[pl.ds(off, L)] = lane + ch

    def issue(i, p, sga, sgb):
      ix = pl.ds(i * ch, ch)
      pltpu.async_copy(a_hbm.at[sidx_all.at[ix]], bufa.at[p], sga)
      pltpu.async_copy(b_hbm.at[didx_all.at[ix]], bufb.at[p], sgb)

    def waitg(p, sga, sgb):
      ix = pl.ds(0, ch)
      pltpu.make_async_copy(a_hbm.at[sidx_all.at[ix]], bufa.at[p], sga).wait()
      pltpu.make_async_copy(b_hbm.at[sidx_all.at[ix]], bufb.at[p], sgb).wait()

    def region(p):
      return zsh.at[pl.ds((s * 2 + p) * ch, ch), :]

    def combine(p, zi):
      pltpu.sync_copy(bufa.at[p], region(p))
      pltpu.sync_copy(bufb.at[p], zsh.at[zi], add=True)

    def write(i, p, sw):
      pltpu.async_copy(region(p), out_hbm.at[pl.ds(base0 + i * ch, ch), :], sw)

    def waitw(p, sw):
      pltpu.make_async_copy(region(p), out_hbm.at[pl.ds(base0, ch), :],
                            sw).wait()

    issue(0, 0, sga0, sgb0)
    issue(1, 1, sga1, sgb1)
    waitg(0, sga0, sgb0)
    combine(0, zi0)
    write(0, 0, sw0)
    issue(2, 0, sga0, sgb0)
    waitg(1, sga1, sgb1)
    combine(1, zi1)
    write(1, 1, sw1)
    issue(3, 1, sga1, sgb1)

    def body(k, carry):
      waitg(0, sga0, sgb0)
      waitw(0, sw0)
      combine(0, zi0)
      write(2 * k, 0, sw0)
      issue(2 * k + 2, 0, sga0, sgb0)
      waitg(1, sga1, sgb1)
      waitw(1, sw1)
      combine(1, zi1)
      write(2 * k + 1, 1, sw1)
      issue(2 * k + 3, 1, sga1, sgb1)
      return carry

    lax.fori_loop(1, (nchunk - 3) // 2, body, 0)
    k0 = nchunk - 3
    waitg(0, sga0, sgb0)
    waitw(0, sw0)
    combine(0, zi0)
    write(k0, 0, sw0)
    issue(k0 + 2, 0, sga0, sgb0)
    waitg(1, sga1, sgb1)
    waitw(1, sw1)
    combine(1, zi1)
    write(k0 + 1, 1, sw1)
    waitg(0, sga0, sgb0)
    waitw(0, sw0)
    combine(0, zi0)
    write(k0 + 2, 0, sw0)
    waitw(0, sw0)
    waitw(1, sw1)

  return _combine


E2 = E // 2
_edge_combine_half = _make_edge_combine(E2, 40)


_BR = 1024
_TE = 6400


def _tc_lin1_body(degp_ref, x_ref, w_ref, g_ref, dinv_ref):
  deg = degp_ref[0] + degp_ref[1] + 1.0
  dv = lax.rsqrt(deg)
  g_ref[...] = dv * jnp.dot(x_ref[...], w_ref[...],
                            preferred_element_type=jnp.float32)
  dinv_ref[...] = dv


def _tc_lin1(degp, xpad, w1):
  grid = NPAD // _BR
  return pl.pallas_call(
      _tc_lin1_body,
      grid=(grid,),
      in_specs=[
          pl.BlockSpec((NC, _BR, 1), lambda i: (0, i, 0)),
          pl.BlockSpec((_BR, D), lambda i: (i, 0)),
          pl.BlockSpec((D, H), lambda i: (0, 0)),
      ],
      out_specs=[
          pl.BlockSpec((_BR, H), lambda i: (i, 0)),
          pl.BlockSpec((_BR, 1), lambda i: (i, 0)),
      ],
      out_shape=[
          jax.ShapeDtypeStruct((NPAD, H), jnp.float32),
          jax.ShapeDtypeStruct((NPAD, 1), jnp.float32),
      ],
  )(degp, xpad, w1)


def _tc_lin2_body(aggp_ref, g_ref, dinv_ref, b_ref, w_ref, out_ref):
  dv = dinv_ref[...]
  h = jax.nn.relu(dv * (aggp_ref[0] + aggp_ref[1] + g_ref[...]) + b_ref[...])
  out_ref[...] = dv * jnp.dot(h, w_ref[...],
                              preferred_element_type=jnp.float32)


def _tc_lin2(aggp, g, dinv, b1, w2):
  grid = NPAD // _BR
  return pl.pallas_call(
      _tc_lin2_body,
      grid=(grid,),
      in_specs=[
          pl.BlockSpec((NC, _BR, H), lambda i: (0, i, 0)),
          pl.BlockSpec((_BR, H), lambda i: (i, 0)),
          pl.BlockSpec((_BR, 1), lambda i: (i, 0)),
          pl.BlockSpec((1, H), lambda i: (0, 0)),
          pl.BlockSpec((H, H), lambda i: (0, 0)),
      ],
      out_specs=pl.BlockSpec((_BR, H), lambda i: (i, 0)),
      out_shape=jax.ShapeDtypeStruct((NPAD, H), jnp.float32),
  )(aggp, g, dinv, b1, w2)


def _tc_tables_body(aggp_ref, g_ref, dinv_ref, b2_ref, ws_ref, wd_ref,
                    bm1_ref, a_ref, bt_ref):
  dv = dinv_ref[...]
  h = dv * (aggp_ref[0] + aggp_ref[1] + g_ref[...]) + b2_ref[...]
  a_ref[...] = jnp.dot(h, ws_ref[...], preferred_element_type=jnp.float32)
  bt_ref[...] = jnp.dot(h, wd_ref[...],
                        preferred_element_type=jnp.float32) + bm1_ref[...]


def _tc_tables(aggp, g, dinv, b2, wm1s, wm1d, bm1):
  grid = NPAD // _BR
  return pl.pallas_call(
      _tc_tables_body,
      grid=(grid,),
      in_specs=[
          pl.BlockSpec((NC, _BR, H), lambda i: (0, i, 0)),
          pl.BlockSpec((_BR, H), lambda i: (i, 0)),
          pl.BlockSpec((_BR, 1), lambda i: (i, 0)),
          pl.BlockSpec((1, H), lambda i: (0, 0)),
          pl.BlockSpec((H, H), lambda i: (0, 0)),
          pl.BlockSpec((H, H), lambda i: (0, 0)),
          pl.BlockSpec((1, H), lambda i: (0, 0)),
      ],
      out_specs=[
          pl.BlockSpec((_BR, H), lambda i: (i, 0)),
          pl.BlockSpec((_BR, H), lambda i: (i, 0)),
      ],
      out_shape=[
          jax.ShapeDtypeStruct((NPAD, H), jnp.float32),
          jax.ShapeDtypeStruct((NPAD, H), jnp.float32),
      ],
  )(aggp, g, dinv, b2, wm1s, wm1d, bm1)


def _tc_mlp_body(zp_ref, eat_ref, wm1e_ref, wm2_ref, bm2_ref,
                 wm3_ref, bm3_ref, out_ref):
  eac = lax.dot_general(eat_ref[...], wm1e_ref[...],
                        (((0,), (0,)), ((), ())),
                        preferred_element_type=jnp.float32)
  z1 = jax.nn.relu(zp_ref[...] + eac)
  z2 = jax.nn.relu(jnp.dot(z1, wm2_ref[...],
                           preferred_element_type=jnp.float32) + bm2_ref[...])
  r = jnp.sum(z2 * wm3_ref[...], axis=1) + bm3_ref[0, 0]
  i = pl.program_id(0)
  out_ref[pl.ds(i * _TE, _TE)] = jax.nn.sigmoid(r)


def _tc_mlp(zp, eat, wm1e, wm2, bm2, wm3row, bm3):
  ne = zp.shape[0]
  grid = ne // _TE
  return pl.pallas_call(
      _tc_mlp_body,
      grid=(grid,),
      in_specs=[
          pl.BlockSpec((_TE, H), lambda i: (i, 0)),
          pl.BlockSpec((DE, _TE), lambda i: (0, i)),
          pl.BlockSpec((DE, H), lambda i: (0, 0)),
          pl.BlockSpec((H, H), lambda i: (0, 0)),
          pl.BlockSpec((1, H), lambda i: (0, 0)),
          pl.BlockSpec((1, H), lambda i: (0, 0)),
          pl.BlockSpec((1, 1), lambda i: (0, 0)),
      ],
      out_specs=pl.BlockSpec((ne,), lambda i: (0,)),
      out_shape=jax.ShapeDtypeStruct((ne,), jnp.float32),
  )(zp, eat, wm1e, wm2, bm2, wm3row, bm3)


def kernel(x, edge_index, edge_attr, W1, b1, W2, b2,
           Wm1, bm1, Wm2, bm2, Wm3, bm3):
  src_f = edge_index[0].reshape(NW, EPW)
  dst_r = edge_index[1].reshape(NW, NCHUNK, CH)
  xpad = jnp.zeros((NPAD, D), jnp.float32).at[:N].set(x)
  zeros_n = jnp.zeros((NPAD,), jnp.float32)
  zeros_nh = jnp.zeros((NPAD, H), jnp.float32)

  degp = _sc_degree(dst_r, zeros_n)
  g0, dinv = _tc_lin1(degp.reshape(NC, NPAD, 1), xpad, W1)
  agg0 = _sc_segsum(src_f, dst_r, g0, zeros_nh)
  g1 = _tc_lin2(agg0, g0, dinv, b1.reshape(1, H), W2)
  agg1 = _sc_segsum(src_f, dst_r, g1, zeros_nh)
  at, bt = _tc_tables(agg1, g1, dinv, b2.reshape(1, H),
                      Wm1[:H], Wm1[H:2 * H], bm1.reshape(1, H))
  eat = edge_attr.T
  wm1e = Wm1[2 * H:]
  bm2r = bm2.reshape(1, H)
  wm3r = Wm3.reshape(1, H)
  bm3r = bm3.reshape(1, 1)
  e2w = E2 // NW
  src_h = edge_index[0].reshape(2, NW, e2w)
  dst_h = edge_index[1].reshape(2, NW, e2w)
  zp0 = _edge_combine_half(src_h[0], dst_h[0], at, bt)
  zp1 = _edge_combine_half(src_h[1], dst_h[1], at, bt)
  out0 = _tc_mlp(zp0, eat[:, :E2], wm1e, Wm2, bm2r, wm3r, bm3r)
  out1 = _tc_mlp(zp1, eat[:, E2:], wm1e, Wm2, bm2r, wm3r, bm3r)
  return jnp.concatenate([out0, out1])


# speedup vs baseline: 1.1025x; 1.1025x over previous
import functools

import jax
import jax.numpy as jnp
from jax import lax
from jax.experimental import pallas as pl
from jax.experimental.pallas import tpu as pltpu
from jax.experimental.pallas import tpu_sc as plsc

N = 10000
E = 320000
D = 128
DE = 4
H = 128

NC, NS, L = 2, 16, 16
NW = NC * NS
EPW = E // NW
CH = 80
NCHUNK = EPW // CH
NPAD = 10240
RPT = NPAD // NS

_mesh = functools.partial(
    plsc.VectorSubcoreMesh, core_axis_name="c", subcore_axis_name="s")


@functools.partial(
    pl.kernel,
    out_type=jax.ShapeDtypeStruct((NC, NPAD), jnp.float32),
    mesh=_mesh(),
    scratch_types=[
        pltpu.VMEM((NCHUNK, CH), jnp.int32),
        pltpu.VMEM((CH,), jnp.float32),
        pltpu.VMEM_SHARED((NPAD,), jnp.float32),
    ],
)
def _sc_degree(dst_hbm, zeros_hbm, out_hbm, didx_all, ones_v, acc_sh):
  c = lax.axis_index("c")
  s = lax.axis_index("s")
  wid = c * NS + s
  for j in range(CH // L):
    ones_v[pl.ds(j * L, L)] = jnp.ones((L,), jnp.float32)
  pltpu.sync_copy(dst_hbm.at[wid], didx_all)
  pltpu.sync_copy(zeros_hbm.at[pl.ds(s * RPT, RPT)],
                  acc_sh.at[pl.ds(s * RPT, RPT)])
  plsc.subcore_barrier()

  def body(i, carry):
    pltpu.sync_copy(ones_v, acc_sh.at[didx_all.at[i]], add=True)
    return carry

  lax.fori_loop(0, NCHUNK, body, 0)
  plsc.subcore_barrier()
  pltpu.sync_copy(acc_sh.at[pl.ds(s * RPT, RPT)],
                  out_hbm.at[c, pl.ds(s * RPT, RPT)])


@functools.partial(
    pl.kernel,
    out_type=jax.ShapeDtypeStruct((NC, NPAD, H), jnp.float32),
    mesh=_mesh(),
    scratch_types=[
        pltpu.VMEM((EPW,), jnp.int32),
        pltpu.VMEM((NCHUNK, CH), jnp.int32),
        pltpu.VMEM((CH, H), jnp.float32),
        pltpu.VMEM((CH, H), jnp.float32),
        pltpu.VMEM_SHARED((NPAD, H), jnp.float32),
        pltpu.SemaphoreType.DMA,
        pltpu.SemaphoreType.DMA,
    ],
)
def _sc_segsum(src_hbm, dst_hbm, g_hbm, zeros_hbm, out_hbm,
               sidx_all, didx_all, rows_a, rows_b, acc_sh, sga, sgb):
  c = lax.axis_index("c")
  s = lax.axis_index("s")
  wid = c * NS + s
  pltpu.sync_copy(src_hbm.at[wid], sidx_all)
  pltpu.sync_copy(dst_hbm.at[wid], didx_all)

  def gat(i, rows, sem):
    return pltpu.make_async_copy(
        g_hbm.at[sidx_all.at[pl.ds(i * CH, CH)]], rows, sem)

  gat(0, rows_a, sga).start()
  gat(1, rows_b, sgb).start()
  pltpu.sync_copy(zeros_hbm.at[pl.ds(s * RPT, RPT), :],
                  acc_sh.at[pl.ds(s * RPT, RPT), :])
  plsc.subcore_barrier()

  def body(k, carry):
    gat(0, rows_a, sga).wait()
    pltpu.sync_copy(rows_a, acc_sh.at[didx_all.at[2 * k]], add=True)
    gat(2 * k + 2, rows_a, sga).start()
    gat(0, rows_b, sgb).wait()
    pltpu.sync_copy(rows_b, acc_sh.at[didx_all.at[2 * k + 1]], add=True)
    gat(2 * k + 3, rows_b, sgb).start()
    return carry

  lax.fori_loop(0, (NCHUNK - 3) // 2, body, 0)
  k0 = NCHUNK - 3
  gat(0, rows_a, sga).wait()
  pltpu.sync_copy(rows_a, acc_sh.at[didx_all.at[k0]], add=True)
  gat(k0 + 2, rows_a, sga).start()
  gat(0, rows_b, sgb).wait()
  pltpu.sync_copy(rows_b, acc_sh.at[didx_all.at[k0 + 1]], add=True)
  gat(0, rows_a, sga).wait()
  pltpu.sync_copy(rows_a, acc_sh.at[didx_all.at[k0 + 2]], add=True)
  plsc.subcore_barrier()
  pltpu.sync_copy(acc_sh.at[pl.ds(s * RPT, RPT), :],
                  out_hbm.at[c, pl.ds(s * RPT, RPT), :])


def _make_edge_combine(ne, ch):
  epw = ne // NW
  nchunk = epw // ch
  assert nchunk % 2 == 1 and nchunk >= 5 and ch % 8 == 0

  @functools.partial(
      pl.kernel,
      out_type=jax.ShapeDtypeStruct((ne, H), jnp.float32),
      mesh=_mesh(),
      scratch_types=[
          pltpu.VMEM((epw,), jnp.int32),
          pltpu.VMEM((epw,), jnp.int32),
          pltpu.VMEM((2, ch, H), jnp.float32),
          pltpu.VMEM((2, ch, H), jnp.float32),
          pltpu.VMEM((ch,), jnp.int32),
          pltpu.VMEM((ch,), jnp.int32),
          pltpu.VMEM_SHARED((NS * 2 * ch, H), jnp.float32),
          pltpu.SemaphoreType.DMA,
          pltpu.SemaphoreType.DMA,
          pltpu.SemaphoreType.DMA,
          pltpu.SemaphoreType.DMA,
          pltpu.SemaphoreType.DMA,
          pltpu.SemaphoreType.DMA,
      ],
  )
  def _combine(src_hbm, dst_hbm, a_hbm, b_hbm, out_hbm,
               sidx_all, didx_all, bufa, bufb, zi0, zi1, zsh,
               sga0, sga1, sgb0, sgb1, sw0, sw1):
    c = lax.axis_index("c")
    s = lax.axis_index("s")
    wid = c * NS + s
    base0 = wid * epw
    pltpu.sync_copy(src_hbm.at[wid], sidx_all)
    pltpu.sync_copy(dst_hbm.at[wid], didx_all)
    offs = list(range(0, ch - L + 1, L))
    if ch % L:
      offs.append(ch - L)
    for off in offs:
      lane = lax.iota(jnp.int32, L) + (s * 2 * ch + off)
      zi0[pl.ds(off, L)] = lane
      zi1[pl.ds(off, L)] = lane + ch

    def issue(i, p, sga, sgb):
      ix = pl.ds(i * ch, ch)
      pltpu.async_copy(a_hbm.at[sidx_all.at[ix]], bufa.at[p], sga)
      pltpu.async_copy(b_hbm.at[didx_all.at[ix]], bufb.at[p], sgb)

    def waitg(p, sga, sgb):
      ix = pl.ds(0, ch)
      pltpu.make_async_copy(a_hbm.at[sidx_all.at[ix]], bufa.at[p], sga).wait()
      pltpu.make_async_copy(b_hbm.at[sidx_all.at[ix]], bufb.at[p], sgb).wait()

    def region(p):
      return zsh.at[pl.ds((s * 2 + p) * ch, ch), :]

    def combine(p, zi):
      pltpu.sync_copy(bufa.at[p], region(p))
      pltpu.sync_copy(bufb.at[p], zsh.at[zi], add=True)

    def write(i, p, sw):
      pltpu.async_copy(region(p), out_hbm.at[pl.ds(base0 + i * ch, ch), :], sw)

    def waitw(p, sw):
      pltpu.make_async_copy(region(p), out_hbm.at[pl.ds(base0, ch), :],
                            sw).wait()

    issue(0, 0, sga0, sgb0)
    issue(1, 1, sga1, sgb1)
    waitg(0, sga0, sgb0)
    combine(0, zi0)
    write(0, 0, sw0)
    issue(2, 0, sga0, sgb0)
    waitg(1, sga1, sgb1)
    combine(1, zi1)
    write(1, 1, sw1)
    issue(3, 1, sga1, sgb1)

    def body(k, carry):
      waitg(0, sga0, sgb0)
      waitw(0, sw0)
      combine(0, zi0)
      write(2 * k, 0, sw0)
      issue(2 * k + 2, 0, sga0, sgb0)
      waitg(1, sga1, sgb1)
      waitw(1, sw1)
      combine(1, zi1)
      write(2 * k + 1, 1, sw1)
      issue(2 * k + 3, 1, sga1, sgb1)
      return carry

    lax.fori_loop(1, (nchunk - 3) // 2, body, 0)
    k0 = nchunk - 3
    waitg(0, sga0, sgb0)
    waitw(0, sw0)
    combine(0, zi0)
    write(k0, 0, sw0)
    issue(k0 + 2, 0, sga0, sgb0)
    waitg(1, sga1, sgb1)
    waitw(1, sw1)
    combine(1, zi1)
    write(k0 + 1, 1, sw1)
    waitg(0, sga0, sgb0)
    waitw(0, sw0)
    combine(0, zi0)
    write(k0 + 2, 0, sw0)
    waitw(0, sw0)
    waitw(1, sw1)

  return _combine


E2 = E // 2
_edge_combine_half = _make_edge_combine(E2, 40)


_BR = 1024
_TE = 6400


def _tc_lin1_body(degp_ref, x_ref, w_ref, g_ref, dinv_ref):
  deg = degp_ref[0] + degp_ref[1] + 1.0
  dv = lax.rsqrt(deg)
  g_ref[...] = dv * jnp.dot(x_ref[...], w_ref[...],
                            preferred_element_type=jnp.float32)
  dinv_ref[...] = dv


def _tc_lin1(degp, xpad, w1):
  grid = NPAD // _BR
  return pl.pallas_call(
      _tc_lin1_body,
      grid=(grid,),
      in_specs=[
          pl.BlockSpec((NC, _BR, 1), lambda i: (0, i, 0)),
          pl.BlockSpec((_BR, D), lambda i: (i, 0)),
          pl.BlockSpec((D, H), lambda i: (0, 0)),
      ],
      out_specs=[
          pl.BlockSpec((_BR, H), lambda i: (i, 0)),
          pl.BlockSpec((_BR, 1), lambda i: (i, 0)),
      ],
      out_shape=[
          jax.ShapeDtypeStruct((NPAD, H), jnp.float32),
          jax.ShapeDtypeStruct((NPAD, 1), jnp.float32),
      ],
  )(degp, xpad, w1)


def _tc_lin2_body(aggp_ref, g_ref, dinv_ref, b_ref, w_ref, out_ref):
  dv = dinv_ref[...]
  h = jax.nn.relu(dv * (aggp_ref[0] + aggp_ref[1] + g_ref[...]) + b_ref[...])
  out_ref[...] = dv * jnp.dot(h, w_ref[...],
                              preferred_element_type=jnp.float32)


def _tc_lin2(aggp, g, dinv, b1, w2):
  grid = NPAD // _BR
  return pl.pallas_call(
      _tc_lin2_body,
      grid=(grid,),
      in_specs=[
          pl.BlockSpec((NC, _BR, H), lambda i: (0, i, 0)),
          pl.BlockSpec((_BR, H), lambda i: (i, 0)),
          pl.BlockSpec((_BR, 1), lambda i: (i, 0)),
          pl.BlockSpec((1, H), lambda i: (0, 0)),
          pl.BlockSpec((H, H), lambda i: (0, 0)),
      ],
      out_specs=pl.BlockSpec((_BR, H), lambda i: (i, 0)),
      out_shape=jax.ShapeDtypeStruct((NPAD, H), jnp.float32),
  )(aggp, g, dinv, b1, w2)


def _tc_tables_body(aggp_ref, g_ref, dinv_ref, b2_ref, ws_ref, wd_ref,
                    bm1_ref, a_ref, bt_ref):
  dv = dinv_ref[...]
  h = dv * (aggp_ref[0] + aggp_ref[1] + g_ref[...]) + b2_ref[...]
  a_ref[...] = jnp.dot(h, ws_ref[...], preferred_element_type=jnp.float32)
  bt_ref[...] = jnp.dot(h, wd_ref[...],
                        preferred_element_type=jnp.float32) + bm1_ref[...]


def _tc_tables(aggp, g, dinv, b2, wm1s, wm1d, bm1):
  grid = NPAD // _BR
  return pl.pallas_call(
      _tc_tables_body,
      grid=(grid,),
      in_specs=[
          pl.BlockSpec((NC, _BR, H), lambda i: (0, i, 0)),
          pl.BlockSpec((_BR, H), lambda i: (i, 0)),
          pl.BlockSpec((_BR, 1), lambda i: (i, 0)),
          pl.BlockSpec((1, H), lambda i: (0, 0)),
          pl.BlockSpec((H, H), lambda i: (0, 0)),
          pl.BlockSpec((H, H), lambda i: (0, 0)),
          pl.BlockSpec((1, H), lambda i: (0, 0)),
      ],
      out_specs=[
          pl.BlockSpec((_BR, H), lambda i: (i, 0)),
          pl.BlockSpec((_BR, H), lambda i: (i, 0)),
      ],
      out_shape=[
          jax.ShapeDtypeStruct((NPAD, H), jnp.float32),
          jax.ShapeDtypeStruct((NPAD, H), jnp.float32),
      ],
  )(aggp, g, dinv, b2, wm1s, wm1d, bm1)


def _tc_mlp_body(zp_ref, eat_ref, wm1e_ref, wm2_ref, bm2_ref,
                 wm3_ref, bm3_ref, out_ref):
  eac = lax.dot_general(eat_ref[...], wm1e_ref[...],
                        (((0,), (0,)), ((), ())),
                        preferred_element_type=jnp.float32)
  z1 = jax.nn.relu(zp_ref[...] + eac)
  z2 = jax.nn.relu(jnp.dot(z1, wm2_ref[...],
                           preferred_element_type=jnp.float32) + bm2_ref[...])
  r = jnp.sum(z2 * wm3_ref[...], axis=1) + bm3_ref[0, 0]
  i = pl.program_id(0)
  out_ref[pl.ds(i * _TE, _TE)] = jax.nn.sigmoid(r)


def _tc_mlp(zp, eat, wm1e, wm2, bm2, wm3row, bm3):
  ne = zp.shape[0]
  grid = ne // _TE
  return pl.pallas_call(
      _tc_mlp_body,
      grid=(grid,),
      in_specs=[
          pl.BlockSpec((_TE, H), lambda i: (i, 0)),
          pl.BlockSpec((DE, _TE), lambda i: (0, i)),
          pl.BlockSpec((DE, H), lambda i: (0, 0)),
          pl.BlockSpec((H, H), lambda i: (0, 0)),
          pl.BlockSpec((1, H), lambda i: (0, 0)),
          pl.BlockSpec((1, H), lambda i: (0, 0)),
          pl.BlockSpec((1, 1), lambda i: (0, 0)),
      ],
      out_specs=pl.BlockSpec((ne,), lambda i: (0,)),
      out_shape=jax.ShapeDtypeStruct((ne,), jnp.float32),
  )(zp, eat, wm1e, wm2, bm2, wm3row, bm3)


def kernel(x, edge_index, edge_attr, W1, b1, W2, b2,
           Wm1, bm1, Wm2, bm2, Wm3, bm3):
  src_f = edge_index[0].reshape(NW, EPW)
  dst_r = edge_index[1].reshape(NW, NCHUNK, CH)
  xpad = jnp.zeros((NPAD, D), jnp.float32).at[:N].set(x)
  zeros_n = jnp.zeros((NPAD,), jnp.float32)
  zeros_nh = jnp.zeros((NPAD, H), jnp.float32)

  degp = _sc_degree(dst_r, zeros_n)
  g0, dinv = _tc_lin1(degp.reshape(NC, NPAD, 1), xpad, W1)
  agg0 = _sc_segsum(src_f, dst_r, g0, zeros_nh)
  g1 = _tc_lin2(agg0, g0, dinv, b1.reshape(1, H), W2)
  agg1 = _sc_segsum(src_f, dst_r, g1, zeros_nh)
  at, bt = _tc_tables(agg1, g1, dinv, b2.reshape(1, H),
                      Wm1[:H], Wm1[H:2 * H], bm1.reshape(1, H))
  eat = edge_attr.T
  wm1e = Wm1[2 * H:]
  bm2r = bm2.reshape(1, H)
  wm3r = Wm3.reshape(1, H)
  bm3r = bm3.reshape(1, 1)
  e2w = E2 // NW
  src_h = edge_index[0].reshape(2, NW, e2w)
  dst_h = edge_index[1].reshape(2, NW, e2w)
  zp0 = _edge_combine_half(src_h[0], dst_h[0], at, bt)
  zp1 = _edge_combine_half(src_h[1], dst_h[1], at, bt)
  out0 = _tc_mlp(zp0, eat[:, :E2], wm1e, Wm2, bm2r, wm3r, bm3r)
  out1 = _tc_mlp(zp1, eat[:, E2:], wm1e, Wm2, bm2r, wm3r, bm3r)
  return jnp.concatenate([out0, out1])
